# Initial kernel scaffold; baseline (speedup 1.0000x reference)
import jax
import jax.numpy as jnp
from jax.experimental import pallas as pl


def kernel(feature, anchors, conv_w, conv_b, obj_w, obj_b, delta_w, delta_b):
    raise NotImplementedError("write your pallas kernel here")


# speedup vs baseline: 1.5760x; 1.5760x over previous
import functools

import jax
import jax.numpy as jnp
import numpy as np
from jax import lax
from jax.experimental import pallas as pl
from jax.experimental.pallas import tpu as pltpu

H = 64
W = 64
A = 3
C = 256
N_PIX = H * W
N_ANCH = N_PIX * A
PRE_NMS = 1000
IMG = 512.0
NMS_THRESH = 0.7
SCALE_CLAMP = float(np.log(1000.0 / 16.0))

_SHIFTS = [(dy, dx) for dy in (-1, 0, 1) for dx in (-1, 0, 1)]


def _conv_head_body(x_ref, w9_ref, cb_ref, hw_ref, hb_ref, out_ref):
    x = x_ref[...]
    col = lax.broadcasted_iota(jnp.int32, (N_PIX, 1), 0) % W
    mask_p = col != (W - 1)
    mask_m = col != 0
    acc = jnp.zeros((N_PIX, C), jnp.float32)
    for k, (dy, dx) in enumerate(_SHIFTS):
        s = W * dy + dx
        if s > 0:
            xs = jnp.concatenate([x[s:], jnp.zeros((s, C), jnp.float32)], axis=0)
        elif s < 0:
            xs = jnp.concatenate([jnp.zeros((-s, C), jnp.float32), x[:s]], axis=0)
        else:
            xs = x
        if dx == 1:
            xs = jnp.where(mask_p, xs, 0.0)
        elif dx == -1:
            xs = jnp.where(mask_m, xs, 0.0)
        acc = acc + jnp.dot(xs, w9_ref[k * C:(k + 1) * C, :],
                            preferred_element_type=jnp.float32)
    t = jax.nn.relu(acc + cb_ref[...])
    out_ref[...] = jnp.dot(t, hw_ref[...], preferred_element_type=jnp.float32) + hb_ref[...]


@jax.jit
def _conv_head(x_t, w9, cb, hw, hb):
    return pl.pallas_call(
        _conv_head_body,
        out_shape=jax.ShapeDtypeStruct((N_PIX, 16), jnp.float32),
    )(x_t, w9, cb, hw, hb)


def kernel(feature, anchors, conv_w, conv_b, obj_w, obj_b, delta_w, delta_b):
    x_t = feature[0].reshape(C, N_PIX).T
    w9 = conv_w.transpose(2, 3, 1, 0).reshape(9 * C, C)
    hw = jnp.zeros((C, 16), jnp.float32)
    hw = hw.at[:, 0:3].set(obj_w[:, :, 0, 0].T)
    hw = hw.at[:, 3:15].set(delta_w[:, :, 0, 0].T)
    hb = jnp.zeros((1, 16), jnp.float32)
    hb = hb.at[0, 0:3].set(obj_b)
    hb = hb.at[0, 3:15].set(delta_b)

    heads = _conv_head(x_t, w9, conv_b.reshape(1, C), hw, hb)

    logits = heads[:, 0:3].T.reshape(1, A, H, W)
    deltas = heads[:, 3:15].T.reshape(1, A * 4, H, W)
    scores = heads[:, 0:3].reshape(-1)
    d = heads[:, 3:15].reshape(-1, 4)

    top_scores, top_idx = lax.top_k(scores, PRE_NMS)
    top_anchors = anchors[top_idx]
    top_d = d[top_idx]
    widths = top_anchors[:, 2] - top_anchors[:, 0]
    heights = top_anchors[:, 3] - top_anchors[:, 1]
    ctr_x = top_anchors[:, 0] + 0.5 * widths
    ctr_y = top_anchors[:, 1] + 0.5 * heights
    dx, dy, dw, dh = top_d[:, 0], top_d[:, 1], top_d[:, 2], top_d[:, 3]
    dw = jnp.clip(dw, None, SCALE_CLAMP)
    dh = jnp.clip(dh, None, SCALE_CLAMP)
    pred_ctr_x = dx * widths + ctr_x
    pred_ctr_y = dy * heights + ctr_y
    pred_w = jnp.exp(dw) * widths
    pred_h = jnp.exp(dh) * heights
    boxes = jnp.stack([pred_ctr_x - 0.5 * pred_w, pred_ctr_y - 0.5 * pred_h,
                       pred_ctr_x + 0.5 * pred_w, pred_ctr_y + 0.5 * pred_h], axis=-1)
    boxes = jnp.clip(boxes, 0.0, IMG)

    area = (boxes[:, 2] - boxes[:, 0]) * (boxes[:, 3] - boxes[:, 1])
    lt = jnp.maximum(boxes[:, None, :2], boxes[None, :, :2])
    rb = jnp.minimum(boxes[:, None, 2:], boxes[None, :, 2:])
    wh = jnp.clip(rb - lt, 0.0, None)
    inter = wh[..., 0] * wh[..., 1]
    union = area[:, None] + area[None, :] - inter
    iou = inter / jnp.maximum(union, 1e-9)
    n = PRE_NMS
    idxs = jnp.arange(n)

    def body(i, supp):
        alive = jnp.logical_not(supp[i])
        row = (iou[i] > NMS_THRESH) & (idxs > i) & alive
        return supp | row

    supp = lax.fori_loop(0, n, body, jnp.zeros((n,), dtype=bool))
    keep_scores = jnp.where(supp, -jnp.inf, top_scores)
    out_scores, out_idx = lax.top_k(keep_scores, n)
    out_boxes = boxes[out_idx]
    return logits, deltas, out_boxes, out_scores


# speedup vs baseline: 17.8803x; 11.3456x over previous
import functools

import jax
import jax.numpy as jnp
import numpy as np
from jax import lax
from jax.experimental import pallas as pl
from jax.experimental.pallas import tpu as pltpu
from jax.experimental.pallas import tpu_sc as plsc

H = 64
W = 64
A = 3
C = 256
N_PIX = H * W
N_ANCH = N_PIX * A
PRE_NMS = 1000
IMG = 512.0
NMS_THRESH = 0.7
SCALE_CLAMP = float(np.log(1000.0 / 16.0))

_SHIFTS = [(dy, dx) for dy in (-1, 0, 1) for dx in (-1, 0, 1)]


def _conv_head_body(x_ref, w9_ref, cb_ref, hw_ref, hb_ref, out_ref):
    x = x_ref[...]
    col = lax.broadcasted_iota(jnp.int32, (N_PIX, 1), 0) % W
    mask_p = col != (W - 1)
    mask_m = col != 0
    acc = jnp.zeros((N_PIX, C), jnp.float32)
    for k, (dy, dx) in enumerate(_SHIFTS):
        s = W * dy + dx
        if s > 0:
            xs = jnp.concatenate([x[s:], jnp.zeros((s, C), jnp.float32)], axis=0)
        elif s < 0:
            xs = jnp.concatenate([jnp.zeros((-s, C), jnp.float32), x[:s]], axis=0)
        else:
            xs = x
        if dx == 1:
            xs = jnp.where(mask_p, xs, 0.0)
        elif dx == -1:
            xs = jnp.where(mask_m, xs, 0.0)
        acc = acc + jnp.dot(xs, w9_ref[k * C:(k + 1) * C, :],
                            preferred_element_type=jnp.float32)
    t = jax.nn.relu(acc + cb_ref[...])
    out_ref[...] = jnp.dot(t, hw_ref[...], preferred_element_type=jnp.float32) + hb_ref[...]


@jax.jit
def _conv_head(x_t, w9, cb, hw, hb):
    return pl.pallas_call(
        _conv_head_body,
        out_shape=jax.ShapeDtypeStruct((N_PIX, 16), jnp.float32),
    )(x_t, w9, cb, hw, hb)


NB = 1024


def _iou_mask_body(rm_ref, cm_ref, out_ref):
    rm = rm_ref[...]
    cm = cm_ref[...]
    x1c, y1c = rm[:, 0:1], rm[:, 1:2]
    x2c, y2c = rm[:, 2:3], rm[:, 3:4]
    x1r, y1r, x2r, y2r = cm[0:1, :], cm[1:2, :], cm[2:3, :], cm[3:4, :]
    area_c = (x2c - x1c) * (y2c - y1c)
    area_r = (x2r - x1r) * (y2r - y1r)
    iw = jnp.clip(jnp.minimum(x2c, x2r) - jnp.maximum(x1c, x1r), 0.0, None)
    ih = jnp.clip(jnp.minimum(y2c, y2r) - jnp.maximum(y1c, y1r), 0.0, None)
    inter = iw * ih
    union = area_c + area_r - inter
    iou = inter / jnp.maximum(union, 1e-9)
    ri = lax.broadcasted_iota(jnp.int32, (NB, NB), 0)
    ci = lax.broadcasted_iota(jnp.int32, (NB, NB), 1)
    m = ((iou > NMS_THRESH) & (ci > ri) & (ri < PRE_NMS) & (ci < PRE_NMS)).astype(jnp.int32)
    bits = lax.broadcasted_iota(jnp.int32, (1, 32), 1)
    cols = []
    for w in range(32):
        block = m[:, w * 32:(w + 1) * 32] << bits
        cols.append(jnp.sum(block, axis=1, keepdims=True))
    out_ref[...] = jnp.concatenate(cols, axis=1)


@jax.jit
def _iou_mask(rm, cm):
    return pl.pallas_call(
        _iou_mask_body,
        out_shape=jax.ShapeDtypeStruct((NB, 32), jnp.int32),
    )(rm, cm)


def _lane_of(vec0, vec1, w):
    l = lax.iota(jnp.int32, 16)
    return (jnp.sum(jnp.where(l == w, vec0, 0))
            + jnp.sum(jnp.where(l == (w - 16), vec1, 0)))


def _nms_sc_body(m_hbm, sc_hbm, cm_hbm, ob_hbm, os_hbm, m_v, sc_v, cm_v, ob_v, os_v):
    c = lax.axis_index("c")
    s = lax.axis_index("s")

    @pl.when(jnp.logical_and(c == 0, s == 0))
    def _():
        pltpu.sync_copy(m_hbm, m_v)
        pltpu.sync_copy(sc_hbm, sc_v)
        pltpu.sync_copy(cm_hbm, cm_v)
        zeros = jnp.zeros((16,), jnp.int32)
        lanes = lax.iota(jnp.int32, 16)

        def step(i, carry):
            s0, s1 = carry
            word = _lane_of(s0, s1, i // 32)
            alive = ((word >> (i % 32)) & 1) == 0
            f = jnp.where(alive, jnp.int32(-1), jnp.int32(0))
            r0 = m_v[pl.ds(i * 32, 16)]
            r1 = m_v[pl.ds(i * 32 + 16, 16)]
            return (s0 | (r0 & f), s1 | (r1 & f))

        s0, s1 = lax.fori_loop(0, PRE_NMS, step, (zeros, zeros))

        def flags_for(g):
            word = _lane_of(s0, s1, g // 2)
            supp = (word >> ((g % 2) * 16 + lanes)) & 1
            valid = (g * 16 + lanes) < PRE_NMS
            alive_f = jnp.where(valid, 1 - supp, 0)
            dead_f = jnp.where(valid, supp, 0)
            return alive_f, dead_f, valid

        def count_step(g, acc):
            alive_f, _, _ = flags_for(g)
            return acc + jnp.sum(alive_f)

        n_alive = lax.fori_loop(0, NB // 16, count_step, jnp.int32(0))

        def scatter_step(g, carry):
            o_a, o_d = carry
            alive_f, dead_f, valid = flags_for(g)
            ca = plsc.cumsum(alive_f)
            cd = plsc.cumsum(dead_f)
            is_alive = alive_f == 1
            pos = jnp.where(is_alive, o_a + ca - 1, o_d + cd - 1)
            sc_g = sc_v[pl.ds(g * 16, 16)]
            val = jnp.where(is_alive, sc_g, -jnp.inf)
            plsc.store_scatter(os_v, [pos], val, mask=valid)
            for cc in range(4):
                coord = cm_v[pl.ds(cc * NB + g * 16, 16)]
                plsc.store_scatter(ob_v, [pos * 4 + cc], coord, mask=valid)
            return (o_a + jnp.sum(alive_f), o_d + jnp.sum(dead_f))

        lax.fori_loop(0, NB // 16, scatter_step, (jnp.int32(0), n_alive))
        pltpu.sync_copy(ob_v, ob_hbm)
        pltpu.sync_copy(os_v, os_hbm)


@jax.jit
def _nms_sc(m_flat, scores_p, cm_flat):
    mesh = plsc.VectorSubcoreMesh(core_axis_name="c", subcore_axis_name="s")
    call = functools.partial(
        pl.kernel,
        mesh=mesh,
        out_type=[jax.ShapeDtypeStruct((PRE_NMS * 4,), jnp.float32),
                  jax.ShapeDtypeStruct((PRE_NMS,), jnp.float32)],
        scratch_types=[pltpu.VMEM((NB * 32,), jnp.int32),
                       pltpu.VMEM((NB,), jnp.float32),
                       pltpu.VMEM((4 * NB,), jnp.float32),
                       pltpu.VMEM((PRE_NMS * 4,), jnp.float32),
                       pltpu.VMEM((PRE_NMS,), jnp.float32)],
        compiler_params=pltpu.CompilerParams(needs_layout_passes=False),
    )(_nms_sc_body)
    return call(m_flat, scores_p, cm_flat)


def kernel(feature, anchors, conv_w, conv_b, obj_w, obj_b, delta_w, delta_b):
    x_t = feature[0].reshape(C, N_PIX).T
    w9 = conv_w.transpose(2, 3, 1, 0).reshape(9 * C, C)
    hw = jnp.zeros((C, 16), jnp.float32)
    hw = hw.at[:, 0:3].set(obj_w[:, :, 0, 0].T)
    hw = hw.at[:, 3:15].set(delta_w[:, :, 0, 0].T)
    hb = jnp.zeros((1, 16), jnp.float32)
    hb = hb.at[0, 0:3].set(obj_b)
    hb = hb.at[0, 3:15].set(delta_b)

    heads = _conv_head(x_t, w9, conv_b.reshape(1, C), hw, hb)

    logits = heads[:, 0:3].T.reshape(1, A, H, W)
    deltas = heads[:, 3:15].T.reshape(1, A * 4, H, W)
    scores = heads[:, 0:3].reshape(-1)
    d = heads[:, 3:15].reshape(-1, 4)

    top_scores, top_idx = lax.top_k(scores, PRE_NMS)
    top_anchors = anchors[top_idx]
    top_d = d[top_idx]
    widths = top_anchors[:, 2] - top_anchors[:, 0]
    heights = top_anchors[:, 3] - top_anchors[:, 1]
    ctr_x = top_anchors[:, 0] + 0.5 * widths
    ctr_y = top_anchors[:, 1] + 0.5 * heights
    dx, dy, dw, dh = top_d[:, 0], top_d[:, 1], top_d[:, 2], top_d[:, 3]
    dw = jnp.clip(dw, None, SCALE_CLAMP)
    dh = jnp.clip(dh, None, SCALE_CLAMP)
    pred_ctr_x = dx * widths + ctr_x
    pred_ctr_y = dy * heights + ctr_y
    pred_w = jnp.exp(dw) * widths
    pred_h = jnp.exp(dh) * heights
    boxes = jnp.stack([pred_ctr_x - 0.5 * pred_w, pred_ctr_y - 0.5 * pred_h,
                       pred_ctr_x + 0.5 * pred_w, pred_ctr_y + 0.5 * pred_h], axis=-1)
    boxes = jnp.clip(boxes, 0.0, IMG)

    boxes_p = jnp.concatenate(
        [boxes, jnp.zeros((NB - PRE_NMS, 4), jnp.float32)], axis=0)
    scores_p = jnp.concatenate(
        [top_scores, jnp.zeros((NB - PRE_NMS,), jnp.float32)], axis=0)
    cm = boxes_p.T
    m = _iou_mask(boxes_p, cm)
    ob_flat, out_scores = _nms_sc(m.reshape(-1), scores_p, cm.reshape(-1))
    out_boxes = ob_flat.reshape(PRE_NMS, 4)
    return logits, deltas, out_boxes, out_scores
